# baseline (speedup 1.0000x reference)
import jax
import jax.numpy as jnp
from jax.experimental import pallas as pl
from jax.experimental.pallas import tpu as pltpu

_ALPHA = 0.2


def _prologue_kernel(x_ref, w_ref, a_ref, hb_ref, p_ref, r_ref,
                     q_ref, t_ref):
    d = w_ref.shape[1]
    h = jnp.dot(x_ref[...], w_ref[...], preferred_element_type=jnp.float32)
    hb_ref[:, :] = jnp.zeros_like(hb_ref)
    hb_ref[:, :d] = h.astype(jnp.bfloat16)
    hb_ref[:, d:d + 1] = jnp.ones((h.shape[0], 1), jnp.bfloat16)
    f = jnp.sum(h * a_ref[0:1, :d], axis=1, keepdims=True)
    g = jnp.sum(h * a_ref[0:1, d:], axis=1, keepdims=True)
    p_ref[...] = jnp.exp(-f)
    r_ref[...] = jnp.exp(-_ALPHA * f)
    q_ref[...] = jnp.exp(-g)
    t_ref[...] = jnp.exp(-_ALPHA * g)


def _main_kernel(adjt_ref, adjb_ref, p_ref, r_ref, qt_ref, tt_ref, hb_ref,
                 out_ref):
    d = out_ref.shape[1]
    bi = adjt_ref.shape[0]

    def half(adj_ref, sl):
        e = (adj_ref[...] * jnp.minimum(p_ref[sl] * qt_ref[...],
                                        r_ref[sl] * tt_ref[...])
             ).astype(jnp.bfloat16)
        acc = jnp.dot(e, hb_ref[...], preferred_element_type=jnp.float32)
        hp = acc[:, :d] / acc[:, d:d + 1]
        return jnp.where(hp > 0, hp, jnp.exp(hp) - 1.0)

    out_ref[:bi] = half(adjt_ref, slice(0, bi))
    out_ref[bi:] = half(adjb_ref, slice(bi, None))


def kernel(x, adj, w, a):
    n, d_in = x.shape
    d = w.shape[1]

    hb, p, r, q, t = pl.pallas_call(
        _prologue_kernel,
        out_shape=(
            jax.ShapeDtypeStruct((n, 2 * d), jnp.bfloat16),
            jax.ShapeDtypeStruct((n, 1), jnp.float32),
            jax.ShapeDtypeStruct((n, 1), jnp.float32),
            jax.ShapeDtypeStruct((n, 1), jnp.float32),
            jax.ShapeDtypeStruct((n, 1), jnp.float32),
        ),
    )(x, w, a)

    qt = q.reshape(1, n)
    tt = t.reshape(1, n)

    bi = 200 if n % 400 == 0 else n
    ni = n // (2 * bi)

    out = pl.pallas_call(
        _main_kernel,
        grid=(ni,),
        in_specs=[
            pl.BlockSpec((bi, n), lambda i: (2 * i, 0)),
            pl.BlockSpec((bi, n), lambda i: (2 * i + 1, 0)),
            pl.BlockSpec((2 * bi, 1), lambda i: (i, 0)),
            pl.BlockSpec((2 * bi, 1), lambda i: (i, 0)),
            pl.BlockSpec((1, n), lambda i: (0, 0)),
            pl.BlockSpec((1, n), lambda i: (0, 0)),
            pl.BlockSpec((n, 2 * d), lambda i: (0, 0)),
        ],
        out_specs=pl.BlockSpec((2 * bi, d), lambda i: (i, 0)),
        out_shape=jax.ShapeDtypeStruct((n, d), jnp.float32),
        compiler_params=pltpu.CompilerParams(
            dimension_semantics=("parallel",)),
    )(adj, adj, p, r, qt, tt, hb)
    return out


# speedup vs baseline: 1.1816x; 1.1816x over previous
import functools

import jax
import jax.numpy as jnp
from jax.experimental import pallas as pl
from jax.experimental.pallas import tpu as pltpu

_ALPHA = 0.2


def _fused_kernel(x_ref, w_ref, a_ref, adj_ref, out_ref,
                  hb_ref, p_ref, r_ref, qt_ref, tt_ref, *, bi):
    i = pl.program_id(0)
    d = w_ref.shape[1]

    @pl.when(i == 0)
    def _prologue():
        h = jnp.dot(x_ref[...], w_ref[...], preferred_element_type=jnp.float32)
        hb_ref[:, :] = jnp.zeros_like(hb_ref)
        hb_ref[:, :d] = h.astype(jnp.bfloat16)
        hb_ref[:, d:d + 1] = jnp.ones((h.shape[0], 1), jnp.bfloat16)
        f = jnp.sum(h * a_ref[0:1, :d], axis=1, keepdims=True)
        p_ref[...] = jnp.exp(-f)
        r_ref[...] = jnp.exp(-_ALPHA * f)
        gt = jax.lax.dot_general(a_ref[:, d:], h, (((1,), (1,)), ((), ())),
                                 preferred_element_type=jnp.float32)
        qt_ref[...] = jnp.exp(-gt)
        tt_ref[...] = jnp.exp(-_ALPHA * gt)

    @pl.when(i > 0)
    def _main():
        k = i - 1
        pb = p_ref[pl.ds(k * bi, bi)]
        rb = r_ref[pl.ds(k * bi, bi)]
        e = (adj_ref[...] * jnp.minimum(pb * qt_ref[...],
                                        rb * tt_ref[...])
             ).astype(jnp.bfloat16)
        acc = jnp.dot(e, hb_ref[...], preferred_element_type=jnp.float32)
        hp = acc[:, :d] / acc[:, d:d + 1]
        out_ref[...] = jnp.where(hp > 0, hp, jnp.exp(hp) - 1.0)


def kernel(x, adj, w, a):
    n, d_in = x.shape
    d = w.shape[1]

    bi = 400 if n % 400 == 0 else n
    ni = n // bi

    out = pl.pallas_call(
        functools.partial(_fused_kernel, bi=bi),
        grid=(ni + 1,),
        in_specs=[
            pl.BlockSpec((n, d_in), lambda i: (0, 0)),
            pl.BlockSpec((d_in, d), lambda i: (0, 0)),
            pl.BlockSpec((1, 2 * d), lambda i: (0, 0)),
            pl.BlockSpec((bi, n),
                         lambda i: (jnp.maximum(i - 1, 0), 0)),
        ],
        out_specs=pl.BlockSpec((bi, d), lambda i: (jnp.maximum(i - 1, 0), 0)),
        out_shape=jax.ShapeDtypeStruct((n, d), jnp.float32),
        scratch_shapes=[
            pltpu.VMEM((n, 2 * d), jnp.bfloat16),
            pltpu.VMEM((n, 1), jnp.float32),
            pltpu.VMEM((n, 1), jnp.float32),
            pltpu.VMEM((1, n), jnp.float32),
            pltpu.VMEM((1, n), jnp.float32),
        ],
    )(x, w, a, adj)
    return out


# speedup vs baseline: 1.1952x; 1.0115x over previous
import functools

import jax
import jax.numpy as jnp
from jax.experimental import pallas as pl
from jax.experimental.pallas import tpu as pltpu

_ALPHA = 0.2


def _fused_kernel(x_ref, w_ref, a_ref, adjt_ref, adjb_ref, out_ref,
                  hb_ref, p_ref, r_ref, qt_ref, tt_ref, *, bi):
    i = pl.program_id(0)
    d = w_ref.shape[1]

    @pl.when(i == 0)
    def _prologue():
        h = jnp.dot(x_ref[...], w_ref[...], preferred_element_type=jnp.float32)
        hb_ref[:, :] = jnp.zeros_like(hb_ref)
        hb_ref[:, :d] = h.astype(jnp.bfloat16)
        hb_ref[:, d:d + 1] = jnp.ones((h.shape[0], 1), jnp.bfloat16)
        f = jnp.sum(h * a_ref[0:1, :d], axis=1, keepdims=True)
        p_ref[...] = jnp.exp(-f)
        r_ref[...] = jnp.exp(-_ALPHA * f)
        gt = jax.lax.dot_general(a_ref[:, d:], h, (((1,), (1,)), ((), ())),
                                 preferred_element_type=jnp.float32)
        qt_ref[...] = jnp.exp(-gt)
        tt_ref[...] = jnp.exp(-_ALPHA * gt)

    @pl.when(i > 0)
    def _main():
        k = i - 1

        def half(adj_half, base, sl):
            pb = p_ref[pl.ds(base, bi)]
            rb = r_ref[pl.ds(base, bi)]
            e = (adj_half * jnp.minimum(pb * qt_ref[...],
                                        rb * tt_ref[...])
                 ).astype(jnp.bfloat16)
            acc = jnp.dot(e, hb_ref[...], preferred_element_type=jnp.float32)
            hp = acc[:, :d] / acc[:, d:d + 1]
            out_ref[sl] = jnp.where(hp > 0, hp, jnp.exp(hp) - 1.0)

        half(adjt_ref[...], k * 2 * bi, slice(0, bi))
        half(adjb_ref[...], k * 2 * bi + bi, slice(bi, None))


def kernel(x, adj, w, a):
    n, d_in = x.shape
    d = w.shape[1]

    bi = 200 if n % 400 == 0 else n
    ni = n // (2 * bi)

    out = pl.pallas_call(
        functools.partial(_fused_kernel, bi=bi),
        grid=(ni + 1,),
        in_specs=[
            pl.BlockSpec((n, d_in), lambda i: (0, 0)),
            pl.BlockSpec((d_in, d), lambda i: (0, 0)),
            pl.BlockSpec((1, 2 * d), lambda i: (0, 0)),
            pl.BlockSpec((bi, n),
                         lambda i: (2 * jnp.maximum(i - 1, 0), 0)),
            pl.BlockSpec((bi, n),
                         lambda i: (2 * jnp.maximum(i - 1, 0) + 1, 0)),
        ],
        out_specs=pl.BlockSpec((2 * bi, d),
                               lambda i: (jnp.maximum(i - 1, 0), 0)),
        out_shape=jax.ShapeDtypeStruct((n, d), jnp.float32),
        scratch_shapes=[
            pltpu.VMEM((n, 2 * d), jnp.bfloat16),
            pltpu.VMEM((n, 1), jnp.float32),
            pltpu.VMEM((n, 1), jnp.float32),
            pltpu.VMEM((1, n), jnp.float32),
            pltpu.VMEM((1, n), jnp.float32),
        ],
    )(x, w, a, adj, adj)
    return out
